# Initial kernel scaffold; baseline (speedup 1.0000x reference)
import jax
import jax.numpy as jnp
from jax.experimental import pallas as pl


def kernel(customer_id, operator_name, user_table, op_table, W1, b1, g1, be1, m1, v1, W2, b2, g2, be2, m2, v2, W3, b3):
    raise NotImplementedError("write your pallas kernel here")


# speedup vs baseline: 1.5161x; 1.5161x over previous
import functools

import jax
import jax.numpy as jnp
from jax import lax
from jax.experimental import pallas as pl
from jax.experimental.pallas import tpu as pltpu
from jax.experimental.pallas import tpu_sc as plsc

B = 16384
EMB = 32
EPS = 1e-3
NC = 2
NS = 16
NW = NC * NS
BPW = B // NW

BLK = 2048


def _sc_gather_body(user_hbm, op_hbm, cid_hbm, oid_hbm, ce_hbm, oe_hbm,
                    idx_u, rows_u, idx_o, rows_o, sem_u, sem_o):
    wid = lax.axis_index("s") * NC + lax.axis_index("c")
    base = wid * BPW
    pltpu.sync_copy(cid_hbm.at[pl.ds(base, BPW)], idx_u)
    pltpu.sync_copy(oid_hbm.at[pl.ds(base, BPW)], idx_o)
    cp_u = pltpu.async_copy(user_hbm.at[idx_u], rows_u, sem_u)
    cp_o = pltpu.async_copy(op_hbm.at[idx_o], rows_o, sem_o)
    cp_u.wait()
    pltpu.sync_copy(rows_u, ce_hbm.at[pl.ds(base, BPW)])
    cp_o.wait()
    pltpu.sync_copy(rows_o, oe_hbm.at[pl.ds(base, BPW)])


def _sc_gather(user_table, op_table, customer_id, operator_name):
    mesh = plsc.VectorSubcoreMesh(core_axis_name="c", subcore_axis_name="s",
                                  num_cores=NC, num_subcores=NS)
    return pl.kernel(
        _sc_gather_body,
        out_type=(jax.ShapeDtypeStruct((B, EMB), jnp.float32),
                  jax.ShapeDtypeStruct((B, EMB), jnp.float32)),
        mesh=mesh,
        scratch_types=[
            pltpu.VMEM((BPW,), jnp.int32),
            pltpu.VMEM((BPW, EMB), jnp.float32),
            pltpu.VMEM((BPW,), jnp.int32),
            pltpu.VMEM((BPW, EMB), jnp.float32),
            pltpu.SemaphoreType.DMA,
            pltpu.SemaphoreType.DMA,
        ],
        compiler_params=pltpu.CompilerParams(use_tc_tiling_on_sc=False),
    )(user_table, op_table, customer_id, operator_name)


def _mlp_body(ce_ref, oe_ref, W1_ref, b1_ref, g1_ref, be1_ref, m1_ref, v1_ref,
              W2_ref, b2_ref, g2_ref, be2_ref, m2_ref, v2_ref,
              W3_ref, b3_ref, out_ref):
    s1 = g1_ref[...] * lax.rsqrt(v1_ref[...] + EPS)
    c1 = (b1_ref[...] - m1_ref[...]) * s1 + be1_ref[...]
    s2 = g2_ref[...] * lax.rsqrt(v2_ref[...] + EPS)
    c2 = (b2_ref[...] - m2_ref[...]) * s2 + be2_ref[...]

    W1 = W1_ref[...] * s1
    acc = jnp.dot(ce_ref[...], W1[:EMB, :],
                  preferred_element_type=jnp.float32)
    acc += jnp.dot(oe_ref[...], W1[EMB:, :],
                   preferred_element_type=jnp.float32)
    h1 = jnp.maximum(acc + c1, 0.0)

    W2 = W2_ref[...] * s2
    h2 = jnp.maximum(jnp.dot(h1, W2, preferred_element_type=jnp.float32) + c2,
                     0.0)

    out_ref[...] = (jnp.dot(h2, W3_ref[...],
                            preferred_element_type=jnp.float32)
                    + b3_ref[...])


def _mlp(ce, oe, W1, b1, g1, be1, m1, v1, W2, b2, g2, be2, m2, v2, W3, b3):
    grid = (B // BLK,)
    full = lambda shape: pl.BlockSpec(shape, lambda i: (0, 0))
    return pl.pallas_call(
        _mlp_body,
        grid=grid,
        in_specs=[
            pl.BlockSpec((BLK, EMB), lambda i: (i, 0)),
            pl.BlockSpec((BLK, EMB), lambda i: (i, 0)),
            full((2 * EMB, 256)), full((1, 256)), full((1, 256)),
            full((1, 256)), full((1, 256)), full((1, 256)),
            full((256, 128)), full((1, 128)), full((1, 128)),
            full((1, 128)), full((1, 128)), full((1, 128)),
            full((128, 1)), full((1, 1)),
        ],
        out_specs=pl.BlockSpec((BLK, 1), lambda i: (i, 0)),
        out_shape=jax.ShapeDtypeStruct((B, 1), jnp.float32),
    )(ce, oe, W1, b1.reshape(1, -1), g1.reshape(1, -1), be1.reshape(1, -1),
      m1.reshape(1, -1), v1.reshape(1, -1), W2, b2.reshape(1, -1),
      g2.reshape(1, -1), be2.reshape(1, -1), m2.reshape(1, -1),
      v2.reshape(1, -1), W3, b3.reshape(1, -1))


def kernel(customer_id, operator_name, user_table, op_table,
           W1, b1, g1, be1, m1, v1, W2, b2, g2, be2, m2, v2, W3, b3):
    ce, oe = _sc_gather(user_table, op_table,
                        customer_id.astype(jnp.int32),
                        operator_name.astype(jnp.int32))
    return _mlp(ce, oe, W1, b1, g1, be1, m1, v1,
                W2, b2, g2, be2, m2, v2, W3, b3)
